# Initial kernel scaffold; baseline (speedup 1.0000x reference)
import jax
import jax.numpy as jnp
from jax.experimental import pallas as pl


def kernel(standard_sample, mean_param, diag_param, post_diag_param, alpha1, alpha2, gamma_param, edge_index):
    raise NotImplementedError("write your pallas kernel here")


# speedup vs baseline: 19.1464x; 19.1464x over previous
import functools

import jax
import jax.numpy as jnp
from jax import lax
from jax.experimental import pallas as pl
from jax.experimental.pallas import tpu as pltpu
from jax.experimental.pallas import tpu_sc as plsc

_LANES = 16
_CHUNK = 128
_NC = 2
_NS = 16
_NW = _NC * _NS


def _sc_edge_pass_body(with_deg, n_pad, e_pad, *refs):
    if with_deg:
        (x_hbm, src_hbm, dst_hbm, zrows_hbm, zdeg_hbm,
         raw_hbm, deg_hbm,
         acc_sh, deg_sh, idx_s, idx_d, rows_v, ones_v, sem) = refs
    else:
        (x_hbm, src_hbm, dst_hbm, zrows_hbm,
         raw_hbm,
         acc_sh, idx_s, idx_d, rows_v, sem) = refs

    cid = lax.axis_index("c")
    sid = lax.axis_index("s")
    rpt = n_pad // _NS

    pltpu.sync_copy(zrows_hbm, acc_sh.at[pl.ds(sid * rpt, rpt)])
    if with_deg:
        pltpu.sync_copy(zdeg_hbm, deg_sh.at[pl.ds(sid * rpt, rpt)])
        for i in range(_CHUNK // 16):
            ones_v[pl.ds(i * 16, 16)] = jnp.ones((16,), jnp.float32)
    plsc.subcore_barrier()

    per_core = e_pad // _NC
    per_sub = per_core // _NS
    chunks = per_sub // _CHUNK
    base = cid * per_core + sid * per_sub

    def body(i, carry):
        off = base + i * _CHUNK
        pltpu.sync_copy(src_hbm.at[pl.ds(off, _CHUNK)], idx_s)
        pltpu.sync_copy(dst_hbm.at[pl.ds(off, _CHUNK)], idx_d)
        pltpu.async_copy(x_hbm.at[idx_s], rows_v, sem).wait()
        pltpu.sync_copy(rows_v, acc_sh.at[idx_d], add=True)
        if with_deg:
            pltpu.sync_copy(ones_v, deg_sh.at[idx_s], add=True)
        return carry

    lax.fori_loop(0, chunks, body, 0)
    plsc.subcore_barrier()

    pltpu.sync_copy(acc_sh.at[pl.ds(sid * rpt, rpt)],
                    raw_hbm.at[cid, pl.ds(sid * rpt, rpt)])
    if with_deg:
        pltpu.sync_copy(deg_sh.at[pl.ds(sid * rpt, rpt)],
                        deg_hbm.at[cid, pl.ds(sid * rpt, rpt)])


def _sc_edge_pass(x_t, src_p, dst_p, n_pad, e_pad, with_deg):
    mesh = plsc.VectorSubcoreMesh(core_axis_name="c", subcore_axis_name="s")
    rpt = n_pad // _NS
    zrows = jnp.zeros((rpt, _LANES), jnp.float32)
    if with_deg:
        out_type = (jax.ShapeDtypeStruct((_NC, n_pad, _LANES), jnp.float32),
                    jax.ShapeDtypeStruct((_NC, n_pad), jnp.float32))
        scratch = [
            pltpu.VMEM_SHARED((n_pad, _LANES), jnp.float32),
            pltpu.VMEM_SHARED((n_pad,), jnp.float32),
            pltpu.VMEM((_CHUNK,), jnp.int32),
            pltpu.VMEM((_CHUNK,), jnp.int32),
            pltpu.VMEM((_CHUNK, _LANES), jnp.float32),
            pltpu.VMEM((_CHUNK,), jnp.float32),
            pltpu.SemaphoreType.DMA,
        ]
        zdeg = jnp.zeros((rpt,), jnp.float32)
        args = (x_t, src_p, dst_p, zrows, zdeg)
    else:
        out_type = jax.ShapeDtypeStruct((_NC, n_pad, _LANES), jnp.float32)
        scratch = [
            pltpu.VMEM_SHARED((n_pad, _LANES), jnp.float32),
            pltpu.VMEM((_CHUNK,), jnp.int32),
            pltpu.VMEM((_CHUNK,), jnp.int32),
            pltpu.VMEM((_CHUNK, _LANES), jnp.float32),
            pltpu.SemaphoreType.DMA,
        ]
        args = (x_t, src_p, dst_p, zrows)

    body = functools.partial(_sc_edge_pass_body, with_deg, n_pad, e_pad)
    return pl.kernel(
        body, out_type=out_type, mesh=mesh, scratch_types=scratch,
        compiler_params=pltpu.CompilerParams(use_tc_tiling_on_sc=False),
    )(*args)


def _prep_body(params_ref, ss_ref, diag_ref, o_ref):
    o_ref[...] = jax.nn.softplus(diag_ref[...]) * ss_ref[...]


def _combine_body(params_ref, x_ref, raw_ref, degp_ref, x1_ref, ld_ref):
    s1 = params_ref[0]
    n1 = params_ref[1]
    g1 = params_ref[2]
    deg = jnp.maximum(degp_ref[0] + degp_ref[1], 1.0)
    ld = jnp.log(deg)
    raw = raw_ref[0] + raw_ref[1]
    x1_ref[...] = (s1 * jnp.exp(g1 * ld)) * x_ref[...] \
        + (n1 * jnp.exp((g1 - 1.0) * ld)) * raw
    ld_ref[...] = ld


def _final_body(params_ref, x_ref, raw_ref, ld_ref, pd_ref, mean_ref, o_ref):
    s2 = params_ref[3]
    n2 = params_ref[4]
    g2 = params_ref[5]
    ld = ld_ref[...]
    raw = raw_ref[0] + raw_ref[1]
    x2 = (s2 * jnp.exp(g2 * ld)) * x_ref[...] \
        + (n2 * jnp.exp((g2 - 1.0) * ld)) * raw
    o_ref[...] = jax.nn.softplus(pd_ref[...]) * x2 + mean_ref[...]


def _ew_call(body, n_pad, bn, out_shapes, params, *arrays):
    grid = n_pad // bn

    def spec_for(a):
        if a.ndim == 3:
            return pl.BlockSpec((2, bn, a.shape[2]), lambda i: (0, i, 0))
        return pl.BlockSpec((bn, a.shape[1]), lambda i: (i, 0))

    in_specs = [pl.BlockSpec(memory_space=pltpu.SMEM)]
    in_specs += [spec_for(a) for a in arrays]
    out_specs = [spec_for(jax.ShapeDtypeStruct(s.shape, s.dtype))
                 for s in out_shapes]
    return pl.pallas_call(
        body,
        grid=(grid,),
        in_specs=in_specs,
        out_specs=out_specs[0] if len(out_shapes) == 1 else out_specs,
        out_shape=out_shapes[0] if len(out_shapes) == 1 else out_shapes,
    )(params, *arrays)


def kernel(standard_sample, mean_param, diag_param, post_diag_param,
           alpha1, alpha2, gamma_param, edge_index):
    S, N = standard_sample.shape
    E = edge_index.shape[1]
    bn = 2048
    n_pad = ((N + 1 + bn - 1) // bn) * bn
    e_pad = ((E + _NW * _CHUNK - 1) // (_NW * _CHUNK)) * (_NW * _CHUNK)

    ss_t = jnp.pad(standard_sample.T, ((0, n_pad - N), (0, _LANES - S)))
    diag_t = jnp.pad(diag_param, (0, n_pad - N)).reshape(n_pad, 1)
    pd_t = jnp.pad(post_diag_param, (0, n_pad - N)).reshape(n_pad, 1)
    mean_t = jnp.pad(mean_param, (0, n_pad - N)).reshape(n_pad, 1)
    src_p = jnp.pad(edge_index[0], (0, e_pad - E), constant_values=N)
    dst_p = jnp.pad(edge_index[1], (0, e_pad - E), constant_values=N)
    sw = jnp.exp(alpha1)
    nw = sw * jnp.tanh(alpha2)
    g = jax.nn.sigmoid(gamma_param)
    params = jnp.stack([sw[0], nw[0], g[0], sw[1], nw[1], g[1]])

    x0 = _ew_call(_prep_body, n_pad, bn,
                  [jax.ShapeDtypeStruct((n_pad, _LANES), jnp.float32)],
                  params, ss_t, diag_t)
    raw1, degp = _sc_edge_pass(x0, src_p, dst_p, n_pad, e_pad, True)
    x1, ld = _ew_call(_combine_body, n_pad, bn,
                      [jax.ShapeDtypeStruct((n_pad, _LANES), jnp.float32),
                       jax.ShapeDtypeStruct((n_pad, 1), jnp.float32)],
                      params, x0, raw1, degp.reshape(_NC, n_pad, 1))
    raw2 = _sc_edge_pass(x1, src_p, dst_p, n_pad, e_pad, False)
    out_t = _ew_call(_final_body, n_pad, bn,
                     [jax.ShapeDtypeStruct((n_pad, _LANES), jnp.float32)],
                     params, x1, raw2, ld, pd_t, mean_t)
    return out_t[:N, :S].T


# speedup vs baseline: 47.6050x; 2.4864x over previous
import functools

import jax
import jax.numpy as jnp
from jax import lax
from jax.experimental import pallas as pl
from jax.experimental.pallas import tpu as pltpu
from jax.experimental.pallas import tpu_sc as plsc

_LANES = 16
_CHUNK = 128
_NC = 2
_NS = 16
_NW = _NC * _NS


_SB = 4


def _sc_edge_pass_body(with_deg, n_pad, e_pad, *refs):
    if with_deg:
        (x_hbm, src_hbm, dst_hbm, zrows_hbm, zdeg_hbm,
         raw_hbm, deg_hbm,
         acc_sh, deg_sh, idx_s, idx_d, rows_v, ones_v,
         sg0, sg1, ss0, ss1) = refs
    else:
        (x_hbm, src_hbm, dst_hbm, zrows_hbm,
         raw_hbm,
         acc_sh, idx_s, idx_d, rows_v,
         sg0, sg1, ss0, ss1) = refs
    sg = (sg0, sg1)
    ss = (ss0, ss1)

    cid = lax.axis_index("c")
    sid = lax.axis_index("s")
    rpt = n_pad // _NS

    pltpu.sync_copy(zrows_hbm, acc_sh.at[pl.ds(sid * rpt, rpt)])
    if with_deg:
        pltpu.sync_copy(zdeg_hbm, deg_sh.at[pl.ds(sid * rpt, rpt)])
        for i in range(_CHUNK // 16):
            ones_v[pl.ds(i * 16, 16)] = jnp.ones((16,), jnp.float32)
    plsc.subcore_barrier()

    rows_per_sub = e_pad // (_NW * _CHUNK)
    nsb = rows_per_sub // _SB
    base_row = (cid * _NS + sid) * rows_per_sub

    def load_idx(b, sbt):
        r0 = base_row + sbt * _SB
        pltpu.sync_copy(src_hbm.at[pl.ds(r0, _SB)], idx_s.at[b])
        pltpu.sync_copy(dst_hbm.at[pl.ds(r0, _SB)], idx_d.at[b])

    def fire_gathers(b):
        for j in range(_SB):
            pltpu.async_copy(x_hbm.at[idx_s.at[b, j]],
                             rows_v.at[b, pl.ds(j * _CHUNK, _CHUNK)], sg[b])

    def drain_gathers(b):
        for j in range(_SB):
            pltpu.make_async_copy(
                x_hbm.at[idx_s.at[b, j]],
                rows_v.at[b, pl.ds(j * _CHUNK, _CHUNK)], sg[b]).wait()

    load_idx(0, 0)
    fire_gathers(0)
    load_idx(1, 1)
    fire_gathers(1)

    def one_sb(b, sbt):
        drain_gathers(b)
        scatters = []
        for j in range(_SB):
            scatters.append(pltpu.async_copy(
                rows_v.at[b, pl.ds(j * _CHUNK, _CHUNK)],
                acc_sh.at[idx_d.at[b, j]], ss[b], add=True))
            if with_deg:
                scatters.append(pltpu.async_copy(
                    ones_v, deg_sh.at[idx_s.at[b, j]], ss[b], add=True))

        for c in scatters:
            c.wait()

        @pl.when(sbt + 2 < nsb)
        def _prefetch():
            load_idx(b, sbt + 2)
            fire_gathers(b)

    def body(g, carry):
        one_sb(0, 2 * g)
        one_sb(1, 2 * g + 1)
        return carry

    lax.fori_loop(0, nsb // 2, body, 0)
    plsc.subcore_barrier()

    pltpu.sync_copy(acc_sh.at[pl.ds(sid * rpt, rpt)],
                    raw_hbm.at[cid, pl.ds(sid * rpt, rpt)])
    if with_deg:
        pltpu.sync_copy(deg_sh.at[pl.ds(sid * rpt, rpt)],
                        deg_hbm.at[cid, pl.ds(sid * rpt, rpt)])


def _sc_edge_pass(x_t, src_p, dst_p, n_pad, e_pad, with_deg):
    mesh = plsc.VectorSubcoreMesh(core_axis_name="c", subcore_axis_name="s")
    rpt = n_pad // _NS
    zrows = jnp.zeros((rpt, _LANES), jnp.float32)
    src2 = src_p.reshape(-1, _CHUNK)
    dst2 = dst_p.reshape(-1, _CHUNK)
    sems = [pltpu.SemaphoreType.DMA] * 4
    if with_deg:
        out_type = (jax.ShapeDtypeStruct((_NC, n_pad, _LANES), jnp.float32),
                    jax.ShapeDtypeStruct((_NC, n_pad), jnp.float32))
        scratch = [
            pltpu.VMEM_SHARED((n_pad, _LANES), jnp.float32),
            pltpu.VMEM_SHARED((n_pad,), jnp.float32),
            pltpu.VMEM((2, _SB, _CHUNK), jnp.int32),
            pltpu.VMEM((2, _SB, _CHUNK), jnp.int32),
            pltpu.VMEM((2, _SB * _CHUNK, _LANES), jnp.float32),
            pltpu.VMEM((_CHUNK,), jnp.float32),
        ] + sems
        zdeg = jnp.zeros((rpt,), jnp.float32)
        args = (x_t, src2, dst2, zrows, zdeg)
    else:
        out_type = jax.ShapeDtypeStruct((_NC, n_pad, _LANES), jnp.float32)
        scratch = [
            pltpu.VMEM_SHARED((n_pad, _LANES), jnp.float32),
            pltpu.VMEM((2, _SB, _CHUNK), jnp.int32),
            pltpu.VMEM((2, _SB, _CHUNK), jnp.int32),
            pltpu.VMEM((2, _SB * _CHUNK, _LANES), jnp.float32),
        ] + sems
        args = (x_t, src2, dst2, zrows)

    body = functools.partial(_sc_edge_pass_body, with_deg, n_pad, e_pad)
    return pl.kernel(
        body, out_type=out_type, mesh=mesh, scratch_types=scratch,
        compiler_params=pltpu.CompilerParams(use_tc_tiling_on_sc=False),
    )(*args)


def _prep_body(params_ref, ss_ref, diag_ref, o_ref):
    o_ref[...] = jax.nn.softplus(diag_ref[...]) * ss_ref[...]


def _combine_body(params_ref, x_ref, raw_ref, degp_ref, x1_ref, ld_ref):
    s1 = params_ref[0]
    n1 = params_ref[1]
    g1 = params_ref[2]
    deg = jnp.maximum(degp_ref[0] + degp_ref[1], 1.0)
    ld = jnp.log(deg)
    raw = raw_ref[0] + raw_ref[1]
    x1_ref[...] = (s1 * jnp.exp(g1 * ld)) * x_ref[...] \
        + (n1 * jnp.exp((g1 - 1.0) * ld)) * raw
    ld_ref[...] = ld


def _final_body(params_ref, x_ref, raw_ref, ld_ref, pd_ref, mean_ref, o_ref):
    s2 = params_ref[3]
    n2 = params_ref[4]
    g2 = params_ref[5]
    ld = ld_ref[...]
    raw = raw_ref[0] + raw_ref[1]
    x2 = (s2 * jnp.exp(g2 * ld)) * x_ref[...] \
        + (n2 * jnp.exp((g2 - 1.0) * ld)) * raw
    o_ref[...] = jax.nn.softplus(pd_ref[...]) * x2 + mean_ref[...]


def _ew_call(body, n_pad, bn, out_shapes, params, *arrays):
    grid = n_pad // bn

    def spec_for(a):
        if a.ndim == 3:
            return pl.BlockSpec((2, bn, a.shape[2]), lambda i: (0, i, 0))
        return pl.BlockSpec((bn, a.shape[1]), lambda i: (i, 0))

    in_specs = [pl.BlockSpec(memory_space=pltpu.SMEM)]
    in_specs += [spec_for(a) for a in arrays]
    out_specs = [spec_for(jax.ShapeDtypeStruct(s.shape, s.dtype))
                 for s in out_shapes]
    return pl.pallas_call(
        body,
        grid=(grid,),
        in_specs=in_specs,
        out_specs=out_specs[0] if len(out_shapes) == 1 else out_specs,
        out_shape=out_shapes[0] if len(out_shapes) == 1 else out_shapes,
    )(params, *arrays)


def kernel(standard_sample, mean_param, diag_param, post_diag_param,
           alpha1, alpha2, gamma_param, edge_index):
    S, N = standard_sample.shape
    E = edge_index.shape[1]
    bn = 2048
    n_pad = ((N + 1 + bn - 1) // bn) * bn
    e_align = _NW * _CHUNK * _SB * 2
    e_pad = ((E + e_align - 1) // e_align) * e_align

    ss_t = jnp.pad(standard_sample.T, ((0, n_pad - N), (0, _LANES - S)))
    diag_t = jnp.pad(diag_param, (0, n_pad - N)).reshape(n_pad, 1)
    pd_t = jnp.pad(post_diag_param, (0, n_pad - N)).reshape(n_pad, 1)
    mean_t = jnp.pad(mean_param, (0, n_pad - N)).reshape(n_pad, 1)
    src_p = jnp.pad(edge_index[0], (0, e_pad - E), constant_values=N)
    dst_p = jnp.pad(edge_index[1], (0, e_pad - E), constant_values=N)
    sw = jnp.exp(alpha1)
    nw = sw * jnp.tanh(alpha2)
    g = jax.nn.sigmoid(gamma_param)
    params = jnp.stack([sw[0], nw[0], g[0], sw[1], nw[1], g[1]])

    x0 = _ew_call(_prep_body, n_pad, bn,
                  [jax.ShapeDtypeStruct((n_pad, _LANES), jnp.float32)],
                  params, ss_t, diag_t)
    raw1, degp = _sc_edge_pass(x0, src_p, dst_p, n_pad, e_pad, True)
    x1, ld = _ew_call(_combine_body, n_pad, bn,
                      [jax.ShapeDtypeStruct((n_pad, _LANES), jnp.float32),
                       jax.ShapeDtypeStruct((n_pad, 1), jnp.float32)],
                      params, x0, raw1, degp.reshape(_NC, n_pad, 1))
    raw2 = _sc_edge_pass(x1, src_p, dst_p, n_pad, e_pad, False)
    out_t = _ew_call(_final_body, n_pad, bn,
                     [jax.ShapeDtypeStruct((n_pad, _LANES), jnp.float32)],
                     params, x1, raw2, ld, pd_t, mean_t)
    return out_t[:N, :S].T


# speedup vs baseline: 54.4102x; 1.1430x over previous
import functools

import jax
import jax.numpy as jnp
from jax import lax
from jax.experimental import pallas as pl
from jax.experimental.pallas import tpu as pltpu
from jax.experimental.pallas import tpu_sc as plsc

_LANES = 16
_CHUNK = 128
_NC = 2
_NS = 16
_NW = _NC * _NS


_WIN = 8


def _sc_edge_pass_body(n_pad, e_pad, x_hbm, src_hbm, dst_hbm, zrows_hbm,
                       raw_hbm, acc_sh, idx_s, idx_d, rows_v, *sems):
    si = sems[0:2]
    sg = sems[2:2 + _WIN]
    ss = sems[2 + _WIN:2 + 2 * _WIN]

    cid = lax.axis_index("c")
    sid = lax.axis_index("s")
    rpt = n_pad // _NS

    pltpu.sync_copy(zrows_hbm, acc_sh.at[pl.ds(sid * rpt, rpt)])
    plsc.subcore_barrier()

    rows_per_sub = e_pad // (_NW * _CHUNK)
    nwin = rows_per_sub // _WIN
    base_row = (cid * _NS + sid) * rows_per_sub

    def fire_idx(h, w):
        r0 = base_row + w * _WIN
        pltpu.async_copy(src_hbm.at[pl.ds(r0, _WIN)], idx_s.at[h], si[h])
        pltpu.async_copy(dst_hbm.at[pl.ds(r0, _WIN)], idx_d.at[h], si[h])

    def wait_idx(h):
        r0 = base_row
        pltpu.make_async_copy(src_hbm.at[pl.ds(r0, _WIN)],
                              idx_s.at[h], si[h]).wait()
        pltpu.make_async_copy(dst_hbm.at[pl.ds(r0, _WIN)],
                              idx_d.at[h], si[h]).wait()

    def drain_bytes(j, sem):
        pltpu.make_async_copy(x_hbm.at[pl.ds(0, _CHUNK)],
                              rows_v.at[j], sem).wait()

    def window(h, w):
        wait_idx(h)
        for j in range(_WIN):
            @pl.when(w >= 1)
            def _drain_ss():
                drain_bytes(j, ss[j])
            pltpu.async_copy(x_hbm.at[idx_s.at[h, j]], rows_v.at[j], sg[j])

        @pl.when(w + 1 < nwin)
        def _prefetch_idx():
            fire_idx(1 - h, w + 1)

        for j in range(_WIN):
            drain_bytes(j, sg[j])
            pltpu.async_copy(rows_v.at[j], acc_sh.at[idx_d.at[h, j]],
                             ss[j], add=True)

    fire_idx(0, 0)

    def body(g, carry):
        window(0, 2 * g)
        window(1, 2 * g + 1)
        return carry

    lax.fori_loop(0, nwin // 2, body, 0)
    for j in range(_WIN):
        drain_bytes(j, ss[j])
    plsc.subcore_barrier()

    pltpu.sync_copy(acc_sh.at[pl.ds(sid * rpt, rpt)],
                    raw_hbm.at[cid, pl.ds(sid * rpt, rpt)])


def _sc_edge_pass(x_t, src2, dst2, n_pad, e_pad):
    mesh = plsc.VectorSubcoreMesh(core_axis_name="c", subcore_axis_name="s")
    rpt = n_pad // _NS
    zrows = jnp.zeros((rpt, _LANES), jnp.float32)
    out_type = jax.ShapeDtypeStruct((_NC, n_pad, _LANES), jnp.float32)
    scratch = [
        pltpu.VMEM_SHARED((n_pad, _LANES), jnp.float32),
        pltpu.VMEM((2, _WIN, _CHUNK), jnp.int32),
        pltpu.VMEM((2, _WIN, _CHUNK), jnp.int32),
        pltpu.VMEM((_WIN, _CHUNK, _LANES), jnp.float32),
    ] + [pltpu.SemaphoreType.DMA] * (2 + 2 * _WIN)
    body = functools.partial(_sc_edge_pass_body, n_pad, e_pad)
    return pl.kernel(
        body, out_type=out_type, mesh=mesh, scratch_types=scratch,
        compiler_params=pltpu.CompilerParams(use_tc_tiling_on_sc=False),
    )(x_t, src2, dst2, zrows)


def _sc_deg_pass_body(n_pad, e_pad, src_hbm, zdeg_hbm, deg_hbm,
                      deg_sh, idx_s, ones_v, *sems):
    si = sems[0:2]
    sd = sems[2:2 + _WIN]

    cid = lax.axis_index("c")
    sid = lax.axis_index("s")
    rpt = n_pad // _NS

    pltpu.sync_copy(zdeg_hbm, deg_sh.at[pl.ds(sid * rpt, rpt)])
    for i in range(_CHUNK // 16):
        ones_v[pl.ds(i * 16, 16)] = jnp.ones((16,), jnp.float32)
    plsc.subcore_barrier()

    rows_per_sub = e_pad // (_NW * _CHUNK)
    nwin = rows_per_sub // _WIN
    base_row = (cid * _NS + sid) * rows_per_sub

    def fire_idx(h, w):
        pltpu.async_copy(src_hbm.at[pl.ds(base_row + w * _WIN, _WIN)],
                         idx_s.at[h], si[h])

    def wait_idx(h):
        pltpu.make_async_copy(src_hbm.at[pl.ds(base_row, _WIN)],
                              idx_s.at[h], si[h]).wait()

    def drain_ones(j):
        pltpu.make_async_copy(src_hbm.at[pl.ds(0, 1)],
                              idx_s.at[0, 0], sd[j]).wait()

    def window(h, w):
        wait_idx(h)
        for j in range(_WIN):
            @pl.when(w >= 1)
            def _drain():
                drain_ones(j)
            pltpu.async_copy(ones_v, deg_sh.at[idx_s.at[h, j]],
                             sd[j], add=True)

        @pl.when(w + 1 < nwin)
        def _prefetch_idx():
            fire_idx(1 - h, w + 1)

    fire_idx(0, 0)

    def body(g, carry):
        window(0, 2 * g)
        window(1, 2 * g + 1)
        return carry

    lax.fori_loop(0, nwin // 2, body, 0)
    for j in range(_WIN):
        drain_ones(j)
    plsc.subcore_barrier()

    pltpu.sync_copy(deg_sh.at[pl.ds(sid * rpt, rpt)],
                    deg_hbm.at[cid, pl.ds(sid * rpt, rpt)])


def _sc_deg_pass(src2, n_pad, e_pad):
    mesh = plsc.VectorSubcoreMesh(core_axis_name="c", subcore_axis_name="s")
    rpt = n_pad // _NS
    zdeg = jnp.zeros((rpt,), jnp.float32)
    out_type = jax.ShapeDtypeStruct((_NC, n_pad), jnp.float32)
    scratch = [
        pltpu.VMEM_SHARED((n_pad,), jnp.float32),
        pltpu.VMEM((2, _WIN, _CHUNK), jnp.int32),
        pltpu.VMEM((_CHUNK,), jnp.float32),
    ] + [pltpu.SemaphoreType.DMA] * (2 + _WIN)
    body = functools.partial(_sc_deg_pass_body, n_pad, e_pad)
    return pl.kernel(
        body, out_type=out_type, mesh=mesh, scratch_types=scratch,
        compiler_params=pltpu.CompilerParams(use_tc_tiling_on_sc=False),
    )(src2, zdeg)


def _prep_body(params_ref, ss_ref, diag_ref, o_ref):
    o_ref[...] = jax.nn.softplus(diag_ref[...]) * ss_ref[...]


def _combine_body(params_ref, x_ref, raw_ref, degp_ref, x1_ref, ld_ref):
    s1 = params_ref[0]
    n1 = params_ref[1]
    g1 = params_ref[2]
    deg = jnp.maximum(degp_ref[0] + degp_ref[1], 1.0)
    ld = jnp.log(deg)
    raw = raw_ref[0] + raw_ref[1]
    x1_ref[...] = (s1 * jnp.exp(g1 * ld)) * x_ref[...] \
        + (n1 * jnp.exp((g1 - 1.0) * ld)) * raw
    ld_ref[...] = ld


def _final_body(params_ref, x_ref, raw_ref, ld_ref, pd_ref, mean_ref, o_ref):
    s2 = params_ref[3]
    n2 = params_ref[4]
    g2 = params_ref[5]
    ld = ld_ref[...]
    raw = raw_ref[0] + raw_ref[1]
    x2 = (s2 * jnp.exp(g2 * ld)) * x_ref[...] \
        + (n2 * jnp.exp((g2 - 1.0) * ld)) * raw
    o_ref[...] = jax.nn.softplus(pd_ref[...]) * x2 + mean_ref[...]


def _ew_call(body, n_pad, bn, out_shapes, params, *arrays):
    grid = n_pad // bn

    def spec_for(a):
        if a.ndim == 3:
            return pl.BlockSpec((2, bn, a.shape[2]), lambda i: (0, i, 0))
        return pl.BlockSpec((bn, a.shape[1]), lambda i: (i, 0))

    in_specs = [pl.BlockSpec(memory_space=pltpu.SMEM)]
    in_specs += [spec_for(a) for a in arrays]
    out_specs = [spec_for(jax.ShapeDtypeStruct(s.shape, s.dtype))
                 for s in out_shapes]
    return pl.pallas_call(
        body,
        grid=(grid,),
        in_specs=in_specs,
        out_specs=out_specs[0] if len(out_shapes) == 1 else out_specs,
        out_shape=out_shapes[0] if len(out_shapes) == 1 else out_shapes,
    )(params, *arrays)


def kernel(standard_sample, mean_param, diag_param, post_diag_param,
           alpha1, alpha2, gamma_param, edge_index):
    S, N = standard_sample.shape
    E = edge_index.shape[1]
    bn = 2048
    n_pad = ((N + 1 + bn - 1) // bn) * bn
    e_align = _NW * _CHUNK * _WIN * 2
    e_pad = ((E + e_align - 1) // e_align) * e_align

    ss_t = jnp.pad(standard_sample.T, ((0, n_pad - N), (0, _LANES - S)))
    diag_t = jnp.pad(diag_param, (0, n_pad - N)).reshape(n_pad, 1)
    pd_t = jnp.pad(post_diag_param, (0, n_pad - N)).reshape(n_pad, 1)
    mean_t = jnp.pad(mean_param, (0, n_pad - N)).reshape(n_pad, 1)
    src2 = jnp.pad(edge_index[0], (0, e_pad - E),
                   constant_values=N).reshape(-1, _CHUNK)
    dst2 = jnp.pad(edge_index[1], (0, e_pad - E),
                   constant_values=N).reshape(-1, _CHUNK)
    sw = jnp.exp(alpha1)
    nw = sw * jnp.tanh(alpha2)
    g = jax.nn.sigmoid(gamma_param)
    params = jnp.stack([sw[0], nw[0], g[0], sw[1], nw[1], g[1]])

    x0 = _ew_call(_prep_body, n_pad, bn,
                  [jax.ShapeDtypeStruct((n_pad, _LANES), jnp.float32)],
                  params, ss_t, diag_t)
    degp = _sc_deg_pass(src2, n_pad, e_pad)
    raw1 = _sc_edge_pass(x0, src2, dst2, n_pad, e_pad)
    x1, ld = _ew_call(_combine_body, n_pad, bn,
                      [jax.ShapeDtypeStruct((n_pad, _LANES), jnp.float32),
                       jax.ShapeDtypeStruct((n_pad, 1), jnp.float32)],
                      params, x0, raw1, degp.reshape(_NC, n_pad, 1))
    raw2 = _sc_edge_pass(x1, src2, dst2, n_pad, e_pad)
    out_t = _ew_call(_final_body, n_pad, bn,
                     [jax.ShapeDtypeStruct((n_pad, _LANES), jnp.float32)],
                     params, x1, raw2, ld, pd_t, mean_t)
    return out_t[:N, :S].T


# speedup vs baseline: 83.0606x; 1.5266x over previous
import functools

import jax
import jax.numpy as jnp
from jax import lax
from jax.experimental import pallas as pl
from jax.experimental.pallas import tpu as pltpu
from jax.experimental.pallas import tpu_sc as plsc

_LANES = 16
_CHUNK = 128
_NC = 2
_NS = 16
_NW = _NC * _NS


_WIN = 8


def _sc_edge_pass_body(n_pad, e_pad, x_hbm, src_hbm, dst_hbm, zrows_hbm,
                       raw_hbm, acc_sh, idx_s, idx_d, rows_v, *sems):
    si = sems[0:2]
    sg = sems[2:2 + _WIN]
    ss = sems[2 + _WIN:2 + 2 * _WIN]

    cid = lax.axis_index("c")
    sid = lax.axis_index("s")
    rpt = n_pad // _NS

    pltpu.sync_copy(zrows_hbm, acc_sh.at[pl.ds(sid * rpt, rpt)])
    plsc.subcore_barrier()

    rows_per_sub = e_pad // (_NW * _CHUNK)
    nwin = rows_per_sub // _WIN
    base_row = (cid * _NS + sid) * rows_per_sub

    def fire_idx(h, w):
        r0 = base_row + w * _WIN
        pltpu.async_copy(src_hbm.at[pl.ds(r0, _WIN)], idx_s.at[h], si[h])
        pltpu.async_copy(dst_hbm.at[pl.ds(r0, _WIN)], idx_d.at[h], si[h])

    def wait_idx(h):
        r0 = base_row
        pltpu.make_async_copy(src_hbm.at[pl.ds(r0, _WIN)],
                              idx_s.at[h], si[h]).wait()
        pltpu.make_async_copy(dst_hbm.at[pl.ds(r0, _WIN)],
                              idx_d.at[h], si[h]).wait()

    def drain_bytes(j, sem):
        pltpu.make_async_copy(x_hbm.at[pl.ds(0, _CHUNK)],
                              rows_v.at[j], sem).wait()

    def window(h, w):
        wait_idx(h)
        for j in range(_WIN):
            @pl.when(w >= 1)
            def _drain_ss():
                drain_bytes(j, ss[j])
            pltpu.async_copy(x_hbm.at[idx_s.at[h, j]], rows_v.at[j], sg[j])

        @pl.when(w + 1 < nwin)
        def _prefetch_idx():
            fire_idx(1 - h, w + 1)

        for j in range(_WIN):
            drain_bytes(j, sg[j])
            pltpu.async_copy(rows_v.at[j], acc_sh.at[idx_d.at[h, j]],
                             ss[j], add=True)

    fire_idx(0, 0)

    def body(g, carry):
        window(0, 2 * g)
        window(1, 2 * g + 1)
        return carry

    lax.fori_loop(0, nwin // 2, body, 0)
    for j in range(_WIN):
        drain_bytes(j, ss[j])
    plsc.subcore_barrier()

    pltpu.sync_copy(acc_sh.at[pl.ds(sid * rpt, rpt)],
                    raw_hbm.at[cid, pl.ds(sid * rpt, rpt)])


def _sc_edge_pass(x_t, src2, dst2, n_pad, e_pad):
    mesh = plsc.VectorSubcoreMesh(core_axis_name="c", subcore_axis_name="s")
    rpt = n_pad // _NS
    zrows = jnp.zeros((rpt, _LANES), jnp.float32)
    out_type = jax.ShapeDtypeStruct((_NC, n_pad, _LANES), jnp.float32)
    scratch = [
        pltpu.VMEM_SHARED((n_pad, _LANES), jnp.float32),
        pltpu.VMEM((2, _WIN, _CHUNK), jnp.int32),
        pltpu.VMEM((2, _WIN, _CHUNK), jnp.int32),
        pltpu.VMEM((_WIN, _CHUNK, _LANES), jnp.float32),
    ] + [pltpu.SemaphoreType.DMA] * (2 + 2 * _WIN)
    body = functools.partial(_sc_edge_pass_body, n_pad, e_pad)
    return pl.kernel(
        body, out_type=out_type, mesh=mesh, scratch_types=scratch,
        compiler_params=pltpu.CompilerParams(use_tc_tiling_on_sc=False),
    )(x_t, src2, dst2, zrows)


def _sc_deg_pass_body(n_pad, e_pad, src_hbm, zdeg_hbm, deg0_hbm, deg1_hbm,
                      deg_sh, idx_s, ones_v, *sems):
    si = sems[0:2]
    sd = sems[2:2 + _WIN]

    cid = lax.axis_index("c")
    sid = lax.axis_index("s")
    rpt = n_pad // _NS

    pltpu.sync_copy(zdeg_hbm, deg_sh.at[pl.ds(sid * rpt, rpt)])
    for i in range(_CHUNK // 16):
        ones_v[pl.ds(i * 16, 16)] = jnp.ones((16,), jnp.float32)
    plsc.subcore_barrier()

    rows_per_sub = e_pad // (_NW * _CHUNK)
    nwin = rows_per_sub // _WIN
    base_row = (cid * _NS + sid) * rows_per_sub

    def fire_idx(h, w):
        pltpu.async_copy(src_hbm.at[pl.ds(base_row + w * _WIN, _WIN)],
                         idx_s.at[h], si[h])

    def wait_idx(h):
        pltpu.make_async_copy(src_hbm.at[pl.ds(base_row, _WIN)],
                              idx_s.at[h], si[h]).wait()

    def drain_ones(j):
        pltpu.make_async_copy(src_hbm.at[pl.ds(0, 1)],
                              idx_s.at[0, 0], sd[j]).wait()

    def window(h, w):
        wait_idx(h)
        for j in range(_WIN):
            @pl.when(w >= 1)
            def _drain():
                drain_ones(j)
            pltpu.async_copy(ones_v, deg_sh.at[idx_s.at[h, j]],
                             sd[j], add=True)

        @pl.when(w + 1 < nwin)
        def _prefetch_idx():
            fire_idx(1 - h, w + 1)

    fire_idx(0, 0)

    def body(g, carry):
        window(0, 2 * g)
        window(1, 2 * g + 1)
        return carry

    lax.fori_loop(0, nwin // 2, body, 0)
    for j in range(_WIN):
        drain_ones(j)
    plsc.subcore_barrier()

    @pl.when(cid == 0)
    def _out0():
        pltpu.sync_copy(deg_sh.at[pl.ds(sid * rpt, rpt)],
                        deg0_hbm.at[pl.ds(sid * rpt, rpt)])

    @pl.when(cid == 1)
    def _out1():
        pltpu.sync_copy(deg_sh.at[pl.ds(sid * rpt, rpt)],
                        deg1_hbm.at[pl.ds(sid * rpt, rpt)])


def _sc_deg_pass(src2, n_pad, e_pad):
    mesh = plsc.VectorSubcoreMesh(core_axis_name="c", subcore_axis_name="s")
    rpt = n_pad // _NS
    zdeg = jnp.zeros((rpt,), jnp.float32)
    out_type = (jax.ShapeDtypeStruct((n_pad,), jnp.float32),
                jax.ShapeDtypeStruct((n_pad,), jnp.float32))
    scratch = [
        pltpu.VMEM_SHARED((n_pad,), jnp.float32),
        pltpu.VMEM((2, _WIN, _CHUNK), jnp.int32),
        pltpu.VMEM((_CHUNK,), jnp.float32),
    ] + [pltpu.SemaphoreType.DMA] * (2 + _WIN)
    body = functools.partial(_sc_deg_pass_body, n_pad, e_pad)
    return pl.kernel(
        body, out_type=out_type, mesh=mesh, scratch_types=scratch,
        compiler_params=pltpu.CompilerParams(use_tc_tiling_on_sc=False),
    )(src2, zdeg)


_CR = 784
_IOTA = None


def _row_idx(r):
    return jnp.full((16,), r, jnp.int32), lax.iota(jnp.int32, 16)


def _sc_prep_body(n_pad, ss_hbm, std_hbm, x0_hbm, ssv, stdv, x0v):
    wid = lax.axis_index("c") * _NS + lax.axis_index("s")
    rows = n_pad // _NW
    base0 = wid * rows

    def chunk(ch, carry):
        base = base0 + ch * _CR
        pltpu.sync_copy(ss_hbm.at[pl.ds(base, _CR)], ssv)
        pltpu.sync_copy(std_hbm.at[pl.ds(base, _CR)], stdv)

        def row(r, c2):
            splat, io = _row_idx(r)
            f = plsc.load_gather(stdv, [splat])
            x = plsc.load_gather(ssv, [splat, io])
            plsc.store_scatter(x0v, [splat, io], f * x)
            return c2

        lax.fori_loop(0, _CR, row, 0, unroll=4)
        pltpu.sync_copy(x0v, x0_hbm.at[pl.ds(base, _CR)])
        return carry

    lax.fori_loop(0, rows // _CR, chunk, 0)


def _sc_prep(ss2, std, n_pad):
    mesh = plsc.VectorSubcoreMesh(core_axis_name="c", subcore_axis_name="s")
    out_type = jax.ShapeDtypeStruct((n_pad, _LANES), jnp.float32)
    scratch = [
        pltpu.VMEM((_CR, _LANES), jnp.float32),
        pltpu.VMEM((_CR,), jnp.float32),
        pltpu.VMEM((_CR, _LANES), jnp.float32),
    ]
    body = functools.partial(_sc_prep_body, n_pad)
    return pl.kernel(
        body, out_type=out_type, mesh=mesh, scratch_types=scratch,
        compiler_params=pltpu.CompilerParams(use_tc_tiling_on_sc=False,
                                            needs_layout_passes=False),
    )(ss2, std)


def _sc_combine_body(n_pad, x_hbm, raw_hbm, a_hbm, b_hbm, xn_hbm,
                     xv, r0v, r1v, av, bv, xnv):
    wid = lax.axis_index("c") * _NS + lax.axis_index("s")
    rows = n_pad // _NW
    base0 = wid * rows

    def chunk(ch, carry):
        base = base0 + ch * _CR
        pltpu.sync_copy(x_hbm.at[pl.ds(base, _CR)], xv)
        pltpu.sync_copy(raw_hbm.at[0, pl.ds(base, _CR)], r0v)
        pltpu.sync_copy(raw_hbm.at[1, pl.ds(base, _CR)], r1v)
        pltpu.sync_copy(a_hbm.at[pl.ds(base, _CR)], av)
        pltpu.sync_copy(b_hbm.at[pl.ds(base, _CR)], bv)

        def row(r, c2):
            splat, io = _row_idx(r)
            fa = plsc.load_gather(av, [splat])
            fb = plsc.load_gather(bv, [splat])
            x = plsc.load_gather(xv, [splat, io])
            r0 = plsc.load_gather(r0v, [splat, io])
            r1 = plsc.load_gather(r1v, [splat, io])
            plsc.store_scatter(xnv, [splat, io], fa * x + fb * (r0 + r1))
            return c2

        lax.fori_loop(0, _CR, row, 0, unroll=4)
        pltpu.sync_copy(xnv, xn_hbm.at[pl.ds(base, _CR)])
        return carry

    lax.fori_loop(0, rows // _CR, chunk, 0)


def _sc_combine(x, raw, a, b, n_pad):
    mesh = plsc.VectorSubcoreMesh(core_axis_name="c", subcore_axis_name="s")
    out_type = jax.ShapeDtypeStruct((n_pad, _LANES), jnp.float32)
    scratch = (
        [pltpu.VMEM((_CR, _LANES), jnp.float32)] * 3
        + [pltpu.VMEM((_CR,), jnp.float32)] * 2
        + [pltpu.VMEM((_CR, _LANES), jnp.float32)]
    )
    body = functools.partial(_sc_combine_body, n_pad)
    return pl.kernel(
        body, out_type=out_type, mesh=mesh, scratch_types=scratch,
        compiler_params=pltpu.CompilerParams(use_tc_tiling_on_sc=False,
                                            needs_layout_passes=False),
    )(x, raw, a, b)


def _sc_final_body(n_pad, x_hbm, raw_hbm, a_hbm, b_hbm, p_hbm, m_hbm, o_hbm,
                   xv, r0v, r1v, av, bv, pv, mv, ov):
    wid = lax.axis_index("c") * _NS + lax.axis_index("s")
    rows = n_pad // _NW
    base0 = wid * rows

    def chunk(ch, carry):
        base = base0 + ch * _CR
        pltpu.sync_copy(x_hbm.at[pl.ds(base, _CR)], xv)
        pltpu.sync_copy(raw_hbm.at[0, pl.ds(base, _CR)], r0v)
        pltpu.sync_copy(raw_hbm.at[1, pl.ds(base, _CR)], r1v)
        pltpu.sync_copy(a_hbm.at[pl.ds(base, _CR)], av)
        pltpu.sync_copy(b_hbm.at[pl.ds(base, _CR)], bv)
        pltpu.sync_copy(p_hbm.at[pl.ds(base, _CR)], pv)
        pltpu.sync_copy(m_hbm.at[pl.ds(base, _CR)], mv)

        def row(r, c2):
            splat, io = _row_idx(r)
            fa = plsc.load_gather(av, [splat])
            fb = plsc.load_gather(bv, [splat])
            fp = plsc.load_gather(pv, [splat])
            fm = plsc.load_gather(mv, [splat])
            x = plsc.load_gather(xv, [splat, io])
            r0 = plsc.load_gather(r0v, [splat, io])
            r1 = plsc.load_gather(r1v, [splat, io])
            x2 = fa * x + fb * (r0 + r1)
            plsc.store_scatter(ov, [splat, io], fp * x2 + fm)
            return c2

        lax.fori_loop(0, _CR, row, 0, unroll=4)
        pltpu.sync_copy(ov, o_hbm.at[pl.ds(base, _CR)])
        return carry

    lax.fori_loop(0, rows // _CR, chunk, 0)


def _sc_final(x, raw, a, b, p, m, n_pad):
    mesh = plsc.VectorSubcoreMesh(core_axis_name="c", subcore_axis_name="s")
    out_type = jax.ShapeDtypeStruct((n_pad, _LANES), jnp.float32)
    scratch = (
        [pltpu.VMEM((_CR, _LANES), jnp.float32)] * 3
        + [pltpu.VMEM((_CR,), jnp.float32)] * 4
        + [pltpu.VMEM((_CR, _LANES), jnp.float32)]
    )
    body = functools.partial(_sc_final_body, n_pad)
    return pl.kernel(
        body, out_type=out_type, mesh=mesh, scratch_types=scratch,
        compiler_params=pltpu.CompilerParams(use_tc_tiling_on_sc=False,
                                            needs_layout_passes=False),
    )(x, raw, a, b, p, m)


def _factor_body(params_ref, deg0_ref, deg1_ref, diag_ref, pdiag_ref,
                 std_ref, a1_ref, b1_ref, a2_ref, b2_ref, pdq_ref):
    s1, n1, g1 = params_ref[0], params_ref[1], params_ref[2]
    s2, n2, g2 = params_ref[3], params_ref[4], params_ref[5]
    deg = jnp.maximum(deg0_ref[...] + deg1_ref[...], 1.0)
    ld = jnp.log(deg)
    a1_ref[...] = s1 * jnp.exp(g1 * ld)
    b1_ref[...] = n1 * jnp.exp((g1 - 1.0) * ld)
    a2_ref[...] = s2 * jnp.exp(g2 * ld)
    b2_ref[...] = n2 * jnp.exp((g2 - 1.0) * ld)
    std_ref[...] = jax.nn.softplus(diag_ref[...])
    pdq_ref[...] = jax.nn.softplus(pdiag_ref[...])


def _tc_factors(params, deg0, deg1, diag_pad, pdiag_pad, n_pad):
    m = n_pad // 128
    shp = jax.ShapeDtypeStruct((m, 128), jnp.float32)
    full = pl.BlockSpec((m, 128), lambda: (0, 0))
    outs = pl.pallas_call(
        _factor_body,
        in_specs=[pl.BlockSpec(memory_space=pltpu.SMEM)] + [full] * 4,
        out_specs=[full] * 6,
        out_shape=[shp] * 6,
    )(params, deg0.reshape(m, 128), deg1.reshape(m, 128),
      diag_pad.reshape(m, 128), pdiag_pad.reshape(m, 128))
    return [o.reshape(n_pad) for o in outs]


def kernel(standard_sample, mean_param, diag_param, post_diag_param,
           alpha1, alpha2, gamma_param, edge_index):
    S, N = standard_sample.shape
    E = edge_index.shape[1]
    bn = 2048
    n_pad = ((N + 1 + bn - 1) // bn) * bn
    e_align = _NW * _CHUNK * _WIN * 2
    e_pad = ((E + e_align - 1) // e_align) * e_align

    ss_t = jnp.pad(standard_sample.T, ((0, n_pad - N), (0, _LANES - S)))
    diag_pad = jnp.pad(diag_param, (0, n_pad - N))
    pdiag_pad = jnp.pad(post_diag_param, (0, n_pad - N))
    mean_pad = jnp.pad(mean_param, (0, n_pad - N))
    src2 = jnp.pad(edge_index[0], (0, e_pad - E),
                   constant_values=N).reshape(-1, _CHUNK)
    dst2 = jnp.pad(edge_index[1], (0, e_pad - E),
                   constant_values=N).reshape(-1, _CHUNK)
    sw = jnp.exp(alpha1)
    nw = sw * jnp.tanh(alpha2)
    g = jax.nn.sigmoid(gamma_param)
    params = jnp.stack([sw[0], nw[0], g[0], sw[1], nw[1], g[1]])

    deg0, deg1 = _sc_deg_pass(src2, n_pad, e_pad)
    std, a1, b1, a2, b2, pdq = _tc_factors(
        params, deg0, deg1, diag_pad, pdiag_pad, n_pad)
    x0 = _sc_prep(ss_t, std, n_pad)
    raw1 = _sc_edge_pass(x0, src2, dst2, n_pad, e_pad)
    x1 = _sc_combine(x0, raw1, a1, b1, n_pad)
    raw2 = _sc_edge_pass(x1, src2, dst2, n_pad, e_pad)
    out_t = _sc_final(x1, raw2, a2, b2, pdq, mean_pad, n_pad)
    return out_t[:N, :S].T


# speedup vs baseline: 85.1222x; 1.0248x over previous
import functools

import jax
import jax.numpy as jnp
from jax import lax
from jax.experimental import pallas as pl
from jax.experimental.pallas import tpu as pltpu
from jax.experimental.pallas import tpu_sc as plsc

_LANES = 16
_CHUNK = 256
_NC = 2
_NS = 16
_NW = _NC * _NS


_WIN = 4


def _sc_edge_pass_body(n_pad, e_pad, x_hbm, src_hbm, dst_hbm, zrows_hbm,
                       raw_hbm, acc_sh, idx_s, idx_d, rows_v, *sems):
    si = sems[0:2]
    sg = sems[2:2 + _WIN]
    ss = sems[2 + _WIN:2 + 2 * _WIN]

    cid = lax.axis_index("c")
    sid = lax.axis_index("s")
    rpt = n_pad // _NS

    pltpu.sync_copy(zrows_hbm, acc_sh.at[pl.ds(sid * rpt, rpt)])
    plsc.subcore_barrier()

    rows_per_sub = e_pad // (_NW * _CHUNK)
    nwin = rows_per_sub // _WIN
    base_row = (cid * _NS + sid) * rows_per_sub

    def fire_idx(h, w):
        r0 = base_row + w * _WIN
        pltpu.async_copy(src_hbm.at[pl.ds(r0, _WIN)], idx_s.at[h], si[h])
        pltpu.async_copy(dst_hbm.at[pl.ds(r0, _WIN)], idx_d.at[h], si[h])

    def wait_idx(h):
        r0 = base_row
        pltpu.make_async_copy(src_hbm.at[pl.ds(r0, _WIN)],
                              idx_s.at[h], si[h]).wait()
        pltpu.make_async_copy(dst_hbm.at[pl.ds(r0, _WIN)],
                              idx_d.at[h], si[h]).wait()

    def drain_bytes(j, sem):
        pltpu.make_async_copy(x_hbm.at[pl.ds(0, _CHUNK)],
                              rows_v.at[j], sem).wait()

    def window(h, w):
        wait_idx(h)
        for j in range(_WIN):
            @pl.when(w >= 1)
            def _drain_ss():
                drain_bytes(j, ss[j])
            pltpu.async_copy(x_hbm.at[idx_s.at[h, j]], rows_v.at[j], sg[j])

        @pl.when(w + 1 < nwin)
        def _prefetch_idx():
            fire_idx(1 - h, w + 1)

        for j in range(_WIN):
            drain_bytes(j, sg[j])
            pltpu.async_copy(rows_v.at[j], acc_sh.at[idx_d.at[h, j]],
                             ss[j], add=True)

    fire_idx(0, 0)

    def body(g, carry):
        window(0, 2 * g)
        window(1, 2 * g + 1)
        return carry

    lax.fori_loop(0, nwin // 2, body, 0)
    for j in range(_WIN):
        drain_bytes(j, ss[j])
    plsc.subcore_barrier()

    pltpu.sync_copy(acc_sh.at[pl.ds(sid * rpt, rpt)],
                    raw_hbm.at[cid, pl.ds(sid * rpt, rpt)])


def _sc_edge_pass(x_t, src2, dst2, n_pad, e_pad):
    mesh = plsc.VectorSubcoreMesh(core_axis_name="c", subcore_axis_name="s")
    rpt = n_pad // _NS
    zrows = jnp.zeros((rpt, _LANES), jnp.float32)
    out_type = jax.ShapeDtypeStruct((_NC, n_pad, _LANES), jnp.float32)
    scratch = [
        pltpu.VMEM_SHARED((n_pad, _LANES), jnp.float32),
        pltpu.VMEM((2, _WIN, _CHUNK), jnp.int32),
        pltpu.VMEM((2, _WIN, _CHUNK), jnp.int32),
        pltpu.VMEM((_WIN, _CHUNK, _LANES), jnp.float32),
    ] + [pltpu.SemaphoreType.DMA] * (2 + 2 * _WIN)
    body = functools.partial(_sc_edge_pass_body, n_pad, e_pad)
    return pl.kernel(
        body, out_type=out_type, mesh=mesh, scratch_types=scratch,
        compiler_params=pltpu.CompilerParams(use_tc_tiling_on_sc=False),
    )(x_t, src2, dst2, zrows)


def _sc_deg_pass_body(n_pad, e_pad, src_hbm, zdeg_hbm, deg0_hbm, deg1_hbm,
                      deg_sh, idx_s, ones_v, *sems):
    si = sems[0:2]
    sd = sems[2:2 + _WIN]

    cid = lax.axis_index("c")
    sid = lax.axis_index("s")
    rpt = n_pad // _NS

    pltpu.sync_copy(zdeg_hbm, deg_sh.at[pl.ds(sid * rpt, rpt)])
    for i in range(_CHUNK // 16):
        ones_v[pl.ds(i * 16, 16)] = jnp.ones((16,), jnp.float32)
    plsc.subcore_barrier()

    rows_per_sub = e_pad // (_NW * _CHUNK)
    nwin = rows_per_sub // _WIN
    base_row = (cid * _NS + sid) * rows_per_sub

    def fire_idx(h, w):
        pltpu.async_copy(src_hbm.at[pl.ds(base_row + w * _WIN, _WIN)],
                         idx_s.at[h], si[h])

    def wait_idx(h):
        pltpu.make_async_copy(src_hbm.at[pl.ds(base_row, _WIN)],
                              idx_s.at[h], si[h]).wait()

    def drain_ones(j):
        pltpu.make_async_copy(src_hbm.at[pl.ds(0, 1)],
                              idx_s.at[0, 0], sd[j]).wait()

    def window(h, w):
        wait_idx(h)
        for j in range(_WIN):
            @pl.when(w >= 1)
            def _drain():
                drain_ones(j)
            pltpu.async_copy(ones_v, deg_sh.at[idx_s.at[h, j]],
                             sd[j], add=True)

        @pl.when(w + 1 < nwin)
        def _prefetch_idx():
            fire_idx(1 - h, w + 1)

    fire_idx(0, 0)

    def body(g, carry):
        window(0, 2 * g)
        window(1, 2 * g + 1)
        return carry

    lax.fori_loop(0, nwin // 2, body, 0)
    for j in range(_WIN):
        drain_ones(j)
    plsc.subcore_barrier()

    @pl.when(cid == 0)
    def _out0():
        pltpu.sync_copy(deg_sh.at[pl.ds(sid * rpt, rpt)],
                        deg0_hbm.at[pl.ds(sid * rpt, rpt)])

    @pl.when(cid == 1)
    def _out1():
        pltpu.sync_copy(deg_sh.at[pl.ds(sid * rpt, rpt)],
                        deg1_hbm.at[pl.ds(sid * rpt, rpt)])


def _sc_deg_pass(src2, n_pad, e_pad):
    mesh = plsc.VectorSubcoreMesh(core_axis_name="c", subcore_axis_name="s")
    rpt = n_pad // _NS
    zdeg = jnp.zeros((rpt,), jnp.float32)
    out_type = (jax.ShapeDtypeStruct((n_pad,), jnp.float32),
                jax.ShapeDtypeStruct((n_pad,), jnp.float32))
    scratch = [
        pltpu.VMEM_SHARED((n_pad,), jnp.float32),
        pltpu.VMEM((2, _WIN, _CHUNK), jnp.int32),
        pltpu.VMEM((_CHUNK,), jnp.float32),
    ] + [pltpu.SemaphoreType.DMA] * (2 + _WIN)
    body = functools.partial(_sc_deg_pass_body, n_pad, e_pad)
    return pl.kernel(
        body, out_type=out_type, mesh=mesh, scratch_types=scratch,
        compiler_params=pltpu.CompilerParams(use_tc_tiling_on_sc=False),
    )(src2, zdeg)


_CR = 784
_IOTA = None


def _row_idx(r):
    return jnp.full((16,), r, jnp.int32), lax.iota(jnp.int32, 16)


def _sc_prep_body(n_pad, ss_hbm, std_hbm, x0_hbm, ssv, stdv, x0v):
    wid = lax.axis_index("c") * _NS + lax.axis_index("s")
    rows = n_pad // _NW
    base0 = wid * rows

    def chunk(ch, carry):
        base = base0 + ch * _CR
        pltpu.sync_copy(ss_hbm.at[pl.ds(base, _CR)], ssv)
        pltpu.sync_copy(std_hbm.at[pl.ds(base, _CR)], stdv)

        def row(r, c2):
            splat, io = _row_idx(r)
            f = plsc.load_gather(stdv, [splat])
            x = plsc.load_gather(ssv, [splat, io])
            plsc.store_scatter(x0v, [splat, io], f * x)
            return c2

        lax.fori_loop(0, _CR, row, 0, unroll=4)
        pltpu.sync_copy(x0v, x0_hbm.at[pl.ds(base, _CR)])
        return carry

    lax.fori_loop(0, rows // _CR, chunk, 0)


def _sc_prep(ss2, std, n_pad):
    mesh = plsc.VectorSubcoreMesh(core_axis_name="c", subcore_axis_name="s")
    out_type = jax.ShapeDtypeStruct((n_pad, _LANES), jnp.float32)
    scratch = [
        pltpu.VMEM((_CR, _LANES), jnp.float32),
        pltpu.VMEM((_CR,), jnp.float32),
        pltpu.VMEM((_CR, _LANES), jnp.float32),
    ]
    body = functools.partial(_sc_prep_body, n_pad)
    return pl.kernel(
        body, out_type=out_type, mesh=mesh, scratch_types=scratch,
        compiler_params=pltpu.CompilerParams(use_tc_tiling_on_sc=False,
                                            needs_layout_passes=False),
    )(ss2, std)


def _sc_combine_body(n_pad, x_hbm, raw_hbm, a_hbm, b_hbm, xn_hbm,
                     xv, r0v, r1v, av, bv, xnv):
    wid = lax.axis_index("c") * _NS + lax.axis_index("s")
    rows = n_pad // _NW
    base0 = wid * rows

    def chunk(ch, carry):
        base = base0 + ch * _CR
        pltpu.sync_copy(x_hbm.at[pl.ds(base, _CR)], xv)
        pltpu.sync_copy(raw_hbm.at[0, pl.ds(base, _CR)], r0v)
        pltpu.sync_copy(raw_hbm.at[1, pl.ds(base, _CR)], r1v)
        pltpu.sync_copy(a_hbm.at[pl.ds(base, _CR)], av)
        pltpu.sync_copy(b_hbm.at[pl.ds(base, _CR)], bv)

        def row(r, c2):
            splat, io = _row_idx(r)
            fa = plsc.load_gather(av, [splat])
            fb = plsc.load_gather(bv, [splat])
            x = plsc.load_gather(xv, [splat, io])
            r0 = plsc.load_gather(r0v, [splat, io])
            r1 = plsc.load_gather(r1v, [splat, io])
            plsc.store_scatter(xnv, [splat, io], fa * x + fb * (r0 + r1))
            return c2

        lax.fori_loop(0, _CR, row, 0, unroll=4)
        pltpu.sync_copy(xnv, xn_hbm.at[pl.ds(base, _CR)])
        return carry

    lax.fori_loop(0, rows // _CR, chunk, 0)


def _sc_combine(x, raw, a, b, n_pad):
    mesh = plsc.VectorSubcoreMesh(core_axis_name="c", subcore_axis_name="s")
    out_type = jax.ShapeDtypeStruct((n_pad, _LANES), jnp.float32)
    scratch = (
        [pltpu.VMEM((_CR, _LANES), jnp.float32)] * 3
        + [pltpu.VMEM((_CR,), jnp.float32)] * 2
        + [pltpu.VMEM((_CR, _LANES), jnp.float32)]
    )
    body = functools.partial(_sc_combine_body, n_pad)
    return pl.kernel(
        body, out_type=out_type, mesh=mesh, scratch_types=scratch,
        compiler_params=pltpu.CompilerParams(use_tc_tiling_on_sc=False,
                                            needs_layout_passes=False),
    )(x, raw, a, b)


def _sc_final_body(n_pad, x_hbm, raw_hbm, a_hbm, b_hbm, p_hbm, m_hbm, o_hbm,
                   xv, r0v, r1v, av, bv, pv, mv, ov):
    wid = lax.axis_index("c") * _NS + lax.axis_index("s")
    rows = n_pad // _NW
    base0 = wid * rows

    def chunk(ch, carry):
        base = base0 + ch * _CR
        pltpu.sync_copy(x_hbm.at[pl.ds(base, _CR)], xv)
        pltpu.sync_copy(raw_hbm.at[0, pl.ds(base, _CR)], r0v)
        pltpu.sync_copy(raw_hbm.at[1, pl.ds(base, _CR)], r1v)
        pltpu.sync_copy(a_hbm.at[pl.ds(base, _CR)], av)
        pltpu.sync_copy(b_hbm.at[pl.ds(base, _CR)], bv)
        pltpu.sync_copy(p_hbm.at[pl.ds(base, _CR)], pv)
        pltpu.sync_copy(m_hbm.at[pl.ds(base, _CR)], mv)

        def row(r, c2):
            splat, io = _row_idx(r)
            fa = plsc.load_gather(av, [splat])
            fb = plsc.load_gather(bv, [splat])
            fp = plsc.load_gather(pv, [splat])
            fm = plsc.load_gather(mv, [splat])
            x = plsc.load_gather(xv, [splat, io])
            r0 = plsc.load_gather(r0v, [splat, io])
            r1 = plsc.load_gather(r1v, [splat, io])
            x2 = fa * x + fb * (r0 + r1)
            plsc.store_scatter(ov, [splat, io], fp * x2 + fm)
            return c2

        lax.fori_loop(0, _CR, row, 0, unroll=4)
        pltpu.sync_copy(ov, o_hbm.at[pl.ds(base, _CR)])
        return carry

    lax.fori_loop(0, rows // _CR, chunk, 0)


def _sc_final(x, raw, a, b, p, m, n_pad):
    mesh = plsc.VectorSubcoreMesh(core_axis_name="c", subcore_axis_name="s")
    out_type = jax.ShapeDtypeStruct((n_pad, _LANES), jnp.float32)
    scratch = (
        [pltpu.VMEM((_CR, _LANES), jnp.float32)] * 3
        + [pltpu.VMEM((_CR,), jnp.float32)] * 4
        + [pltpu.VMEM((_CR, _LANES), jnp.float32)]
    )
    body = functools.partial(_sc_final_body, n_pad)
    return pl.kernel(
        body, out_type=out_type, mesh=mesh, scratch_types=scratch,
        compiler_params=pltpu.CompilerParams(use_tc_tiling_on_sc=False,
                                            needs_layout_passes=False),
    )(x, raw, a, b, p, m)


def _factor_body(params_ref, deg0_ref, deg1_ref, diag_ref, pdiag_ref,
                 std_ref, a1_ref, b1_ref, a2_ref, b2_ref, pdq_ref):
    s1, n1, g1 = params_ref[0], params_ref[1], params_ref[2]
    s2, n2, g2 = params_ref[3], params_ref[4], params_ref[5]
    deg = jnp.maximum(deg0_ref[...] + deg1_ref[...], 1.0)
    ld = jnp.log(deg)
    a1_ref[...] = s1 * jnp.exp(g1 * ld)
    b1_ref[...] = n1 * jnp.exp((g1 - 1.0) * ld)
    a2_ref[...] = s2 * jnp.exp(g2 * ld)
    b2_ref[...] = n2 * jnp.exp((g2 - 1.0) * ld)
    std_ref[...] = jax.nn.softplus(diag_ref[...])
    pdq_ref[...] = jax.nn.softplus(pdiag_ref[...])


def _tc_factors(params, deg0, deg1, diag_pad, pdiag_pad, n_pad):
    m = n_pad // 128
    shp = jax.ShapeDtypeStruct((m, 128), jnp.float32)
    full = pl.BlockSpec((m, 128), lambda: (0, 0))
    outs = pl.pallas_call(
        _factor_body,
        in_specs=[pl.BlockSpec(memory_space=pltpu.SMEM)] + [full] * 4,
        out_specs=[full] * 6,
        out_shape=[shp] * 6,
    )(params, deg0.reshape(m, 128), deg1.reshape(m, 128),
      diag_pad.reshape(m, 128), pdiag_pad.reshape(m, 128))
    return [o.reshape(n_pad) for o in outs]


def kernel(standard_sample, mean_param, diag_param, post_diag_param,
           alpha1, alpha2, gamma_param, edge_index):
    S, N = standard_sample.shape
    E = edge_index.shape[1]
    bn = 2048
    n_pad = ((N + 1 + bn - 1) // bn) * bn
    e_align = _NW * _CHUNK * _WIN * 2
    e_pad = ((E + e_align - 1) // e_align) * e_align

    ss_t = jnp.pad(standard_sample.T, ((0, n_pad - N), (0, _LANES - S)))
    diag_pad = jnp.pad(diag_param, (0, n_pad - N))
    pdiag_pad = jnp.pad(post_diag_param, (0, n_pad - N))
    mean_pad = jnp.pad(mean_param, (0, n_pad - N))
    src2 = jnp.pad(edge_index[0], (0, e_pad - E),
                   constant_values=N).reshape(-1, _CHUNK)
    dst2 = jnp.pad(edge_index[1], (0, e_pad - E),
                   constant_values=N).reshape(-1, _CHUNK)
    sw = jnp.exp(alpha1)
    nw = sw * jnp.tanh(alpha2)
    g = jax.nn.sigmoid(gamma_param)
    params = jnp.stack([sw[0], nw[0], g[0], sw[1], nw[1], g[1]])

    deg0, deg1 = _sc_deg_pass(src2, n_pad, e_pad)
    std, a1, b1, a2, b2, pdq = _tc_factors(
        params, deg0, deg1, diag_pad, pdiag_pad, n_pad)
    x0 = _sc_prep(ss_t, std, n_pad)
    raw1 = _sc_edge_pass(x0, src2, dst2, n_pad, e_pad)
    x1 = _sc_combine(x0, raw1, a1, b1, n_pad)
    raw2 = _sc_edge_pass(x1, src2, dst2, n_pad, e_pad)
    out_t = _sc_final(x1, raw2, a2, b2, pdq, mean_pad, n_pad)
    return out_t[:N, :S].T


# speedup vs baseline: 88.1548x; 1.0356x over previous
import functools

import jax
import jax.numpy as jnp
from jax import lax
from jax.experimental import pallas as pl
from jax.experimental.pallas import tpu as pltpu
from jax.experimental.pallas import tpu_sc as plsc

_LANES = 16
_CHUNK = 256
_NC = 2
_NS = 16
_NW = _NC * _NS


_WIN = 4


def _sc_edge_pass_body(with_deg, n_pad, e_pad, *refs):
    if with_deg:
        (x_hbm, src_hbm, dst_hbm, zrows_hbm, zdeg_hbm,
         raw_hbm, deg0_hbm, deg1_hbm,
         acc_sh, deg_sh, idx_s, idx_d, rows_v, ones_v, *sems) = refs
    else:
        (x_hbm, src_hbm, dst_hbm, zrows_hbm,
         raw_hbm,
         acc_sh, idx_s, idx_d, rows_v, *sems) = refs
    si = sems[0:2]
    sg = sems[2:2 + _WIN]
    ss = sems[2 + _WIN:2 + 2 * _WIN]

    cid = lax.axis_index("c")
    sid = lax.axis_index("s")
    rpt = n_pad // _NS

    pltpu.sync_copy(zrows_hbm, acc_sh.at[pl.ds(sid * rpt, rpt)])
    if with_deg:
        pltpu.sync_copy(zdeg_hbm, deg_sh.at[pl.ds(sid * rpt, rpt)])
        for i in range(_CHUNK // 16):
            ones_v[pl.ds(i * 16, 16)] = jnp.ones((16,), jnp.float32)
    plsc.subcore_barrier()

    rows_per_sub = e_pad // (_NW * _CHUNK)
    nwin = rows_per_sub // _WIN
    base_row = (cid * _NS + sid) * rows_per_sub

    def fire_idx(h, w):
        r0 = base_row + w * _WIN
        pltpu.async_copy(src_hbm.at[pl.ds(r0, _WIN)], idx_s.at[h], si[h])
        pltpu.async_copy(dst_hbm.at[pl.ds(r0, _WIN)], idx_d.at[h], si[h])

    def wait_idx(h):
        r0 = base_row
        pltpu.make_async_copy(src_hbm.at[pl.ds(r0, _WIN)],
                              idx_s.at[h], si[h]).wait()
        pltpu.make_async_copy(dst_hbm.at[pl.ds(r0, _WIN)],
                              idx_d.at[h], si[h]).wait()

    def drain_bytes(j, sem):
        pltpu.make_async_copy(x_hbm.at[pl.ds(0, _CHUNK)],
                              rows_v.at[j], sem).wait()

    def drain_ones(j, sem):
        pltpu.make_async_copy(src_hbm.at[pl.ds(0, 1)],
                              idx_s.at[0, 0], sem).wait()

    def window(h, w):
        wait_idx(h)
        for j in range(_WIN):
            @pl.when(w >= 1)
            def _drain_ss():
                drain_bytes(j, ss[j])
                if with_deg:
                    drain_ones(j, ss[j])
            pltpu.async_copy(x_hbm.at[idx_s.at[h, j]], rows_v.at[j], sg[j])

        @pl.when(w + 1 < nwin)
        def _prefetch_idx():
            fire_idx(1 - h, w + 1)

        for j in range(_WIN):
            drain_bytes(j, sg[j])
            pltpu.async_copy(rows_v.at[j], acc_sh.at[idx_d.at[h, j]],
                             ss[j], add=True)
            if with_deg:
                pltpu.async_copy(ones_v, deg_sh.at[idx_s.at[h, j]],
                                 ss[j], add=True)

    fire_idx(0, 0)

    def body(g, carry):
        window(0, 2 * g)
        window(1, 2 * g + 1)
        return carry

    lax.fori_loop(0, nwin // 2, body, 0)
    for j in range(_WIN):
        drain_bytes(j, ss[j])
        if with_deg:
            drain_ones(j, ss[j])
    plsc.subcore_barrier()

    pltpu.sync_copy(acc_sh.at[pl.ds(sid * rpt, rpt)],
                    raw_hbm.at[cid, pl.ds(sid * rpt, rpt)])
    if with_deg:
        @pl.when(cid == 0)
        def _out0():
            pltpu.sync_copy(deg_sh.at[pl.ds(sid * rpt, rpt)],
                            deg0_hbm.at[pl.ds(sid * rpt, rpt)])

        @pl.when(cid == 1)
        def _out1():
            pltpu.sync_copy(deg_sh.at[pl.ds(sid * rpt, rpt)],
                            deg1_hbm.at[pl.ds(sid * rpt, rpt)])


def _sc_edge_pass(x_t, src2, dst2, n_pad, e_pad, with_deg=False):
    mesh = plsc.VectorSubcoreMesh(core_axis_name="c", subcore_axis_name="s")
    rpt = n_pad // _NS
    zrows = jnp.zeros((rpt, _LANES), jnp.float32)
    raw_t = jax.ShapeDtypeStruct((_NC, n_pad, _LANES), jnp.float32)
    idx_scr = [
        pltpu.VMEM((2, _WIN, _CHUNK), jnp.int32),
        pltpu.VMEM((2, _WIN, _CHUNK), jnp.int32),
        pltpu.VMEM((_WIN, _CHUNK, _LANES), jnp.float32),
    ]
    sems = [pltpu.SemaphoreType.DMA] * (2 + 2 * _WIN)
    if with_deg:
        out_type = (raw_t,
                    jax.ShapeDtypeStruct((n_pad,), jnp.float32),
                    jax.ShapeDtypeStruct((n_pad,), jnp.float32))
        scratch = ([pltpu.VMEM_SHARED((n_pad, _LANES), jnp.float32),
                    pltpu.VMEM_SHARED((n_pad,), jnp.float32)]
                   + idx_scr + [pltpu.VMEM((_CHUNK,), jnp.float32)] + sems)
        zdeg = jnp.zeros((rpt,), jnp.float32)
        args = (x_t, src2, dst2, zrows, zdeg)
    else:
        out_type = raw_t
        scratch = ([pltpu.VMEM_SHARED((n_pad, _LANES), jnp.float32)]
                   + idx_scr + sems)
        args = (x_t, src2, dst2, zrows)
    body = functools.partial(_sc_edge_pass_body, with_deg, n_pad, e_pad)
    return pl.kernel(
        body, out_type=out_type, mesh=mesh, scratch_types=scratch,
        compiler_params=pltpu.CompilerParams(use_tc_tiling_on_sc=False),
    )(*args)


_CR = 784
_IOTA = None


def _row_idx(r):
    return jnp.full((16,), r, jnp.int32), lax.iota(jnp.int32, 16)


def _sc_prep_body(n_pad, ss_hbm, std_hbm, x0_hbm, ssv, stdv, x0v):
    wid = lax.axis_index("c") * _NS + lax.axis_index("s")
    rows = n_pad // _NW
    base0 = wid * rows

    def chunk(ch, carry):
        base = base0 + ch * _CR
        pltpu.sync_copy(ss_hbm.at[pl.ds(base, _CR)], ssv)
        pltpu.sync_copy(std_hbm.at[pl.ds(base, _CR)], stdv)

        def row(r, c2):
            splat, io = _row_idx(r)
            f = plsc.load_gather(stdv, [splat])
            x = plsc.load_gather(ssv, [splat, io])
            plsc.store_scatter(x0v, [splat, io], f * x)
            return c2

        lax.fori_loop(0, _CR, row, 0, unroll=4)
        pltpu.sync_copy(x0v, x0_hbm.at[pl.ds(base, _CR)])
        return carry

    lax.fori_loop(0, rows // _CR, chunk, 0)


def _sc_prep(ss2, std, n_pad):
    mesh = plsc.VectorSubcoreMesh(core_axis_name="c", subcore_axis_name="s")
    out_type = jax.ShapeDtypeStruct((n_pad, _LANES), jnp.float32)
    scratch = [
        pltpu.VMEM((_CR, _LANES), jnp.float32),
        pltpu.VMEM((_CR,), jnp.float32),
        pltpu.VMEM((_CR, _LANES), jnp.float32),
    ]
    body = functools.partial(_sc_prep_body, n_pad)
    return pl.kernel(
        body, out_type=out_type, mesh=mesh, scratch_types=scratch,
        compiler_params=pltpu.CompilerParams(use_tc_tiling_on_sc=False,
                                            needs_layout_passes=False),
    )(ss2, std)


def _sc_combine_body(n_pad, x_hbm, raw_hbm, a_hbm, b_hbm, xn_hbm,
                     xv, r0v, r1v, av, bv, xnv):
    wid = lax.axis_index("c") * _NS + lax.axis_index("s")
    rows = n_pad // _NW
    base0 = wid * rows

    def chunk(ch, carry):
        base = base0 + ch * _CR
        pltpu.sync_copy(x_hbm.at[pl.ds(base, _CR)], xv)
        pltpu.sync_copy(raw_hbm.at[0, pl.ds(base, _CR)], r0v)
        pltpu.sync_copy(raw_hbm.at[1, pl.ds(base, _CR)], r1v)
        pltpu.sync_copy(a_hbm.at[pl.ds(base, _CR)], av)
        pltpu.sync_copy(b_hbm.at[pl.ds(base, _CR)], bv)

        def row(r, c2):
            splat, io = _row_idx(r)
            fa = plsc.load_gather(av, [splat])
            fb = plsc.load_gather(bv, [splat])
            x = plsc.load_gather(xv, [splat, io])
            r0 = plsc.load_gather(r0v, [splat, io])
            r1 = plsc.load_gather(r1v, [splat, io])
            plsc.store_scatter(xnv, [splat, io], fa * x + fb * (r0 + r1))
            return c2

        lax.fori_loop(0, _CR, row, 0, unroll=4)
        pltpu.sync_copy(xnv, xn_hbm.at[pl.ds(base, _CR)])
        return carry

    lax.fori_loop(0, rows // _CR, chunk, 0)


def _sc_combine(x, raw, a, b, n_pad):
    mesh = plsc.VectorSubcoreMesh(core_axis_name="c", subcore_axis_name="s")
    out_type = jax.ShapeDtypeStruct((n_pad, _LANES), jnp.float32)
    scratch = (
        [pltpu.VMEM((_CR, _LANES), jnp.float32)] * 3
        + [pltpu.VMEM((_CR,), jnp.float32)] * 2
        + [pltpu.VMEM((_CR, _LANES), jnp.float32)]
    )
    body = functools.partial(_sc_combine_body, n_pad)
    return pl.kernel(
        body, out_type=out_type, mesh=mesh, scratch_types=scratch,
        compiler_params=pltpu.CompilerParams(use_tc_tiling_on_sc=False,
                                            needs_layout_passes=False),
    )(x, raw, a, b)


def _sc_final_body(n_pad, x_hbm, raw_hbm, a_hbm, b_hbm, p_hbm, m_hbm, o_hbm,
                   xv, r0v, r1v, av, bv, pv, mv, ov):
    wid = lax.axis_index("c") * _NS + lax.axis_index("s")
    rows = n_pad // _NW
    base0 = wid * rows

    def chunk(ch, carry):
        base = base0 + ch * _CR
        pltpu.sync_copy(x_hbm.at[pl.ds(base, _CR)], xv)
        pltpu.sync_copy(raw_hbm.at[0, pl.ds(base, _CR)], r0v)
        pltpu.sync_copy(raw_hbm.at[1, pl.ds(base, _CR)], r1v)
        pltpu.sync_copy(a_hbm.at[pl.ds(base, _CR)], av)
        pltpu.sync_copy(b_hbm.at[pl.ds(base, _CR)], bv)
        pltpu.sync_copy(p_hbm.at[pl.ds(base, _CR)], pv)
        pltpu.sync_copy(m_hbm.at[pl.ds(base, _CR)], mv)

        def row(r, c2):
            splat, io = _row_idx(r)
            fa = plsc.load_gather(av, [splat])
            fb = plsc.load_gather(bv, [splat])
            fp = plsc.load_gather(pv, [splat])
            fm = plsc.load_gather(mv, [splat])
            x = plsc.load_gather(xv, [splat, io])
            r0 = plsc.load_gather(r0v, [splat, io])
            r1 = plsc.load_gather(r1v, [splat, io])
            x2 = fa * x + fb * (r0 + r1)
            plsc.store_scatter(ov, [splat, io], fp * x2 + fm)
            return c2

        lax.fori_loop(0, _CR, row, 0, unroll=4)
        pltpu.sync_copy(ov, o_hbm.at[pl.ds(base, _CR)])
        return carry

    lax.fori_loop(0, rows // _CR, chunk, 0)


def _sc_final(x, raw, a, b, p, m, n_pad):
    mesh = plsc.VectorSubcoreMesh(core_axis_name="c", subcore_axis_name="s")
    out_type = jax.ShapeDtypeStruct((n_pad, _LANES), jnp.float32)
    scratch = (
        [pltpu.VMEM((_CR, _LANES), jnp.float32)] * 3
        + [pltpu.VMEM((_CR,), jnp.float32)] * 4
        + [pltpu.VMEM((_CR, _LANES), jnp.float32)]
    )
    body = functools.partial(_sc_final_body, n_pad)
    return pl.kernel(
        body, out_type=out_type, mesh=mesh, scratch_types=scratch,
        compiler_params=pltpu.CompilerParams(use_tc_tiling_on_sc=False,
                                            needs_layout_passes=False),
    )(x, raw, a, b, p, m)


def _factor_body(params_ref, deg0_ref, deg1_ref, diag_ref, pdiag_ref,
                 std_ref, a1_ref, b1_ref, a2_ref, b2_ref, pdq_ref):
    s1, n1, g1 = params_ref[0], params_ref[1], params_ref[2]
    s2, n2, g2 = params_ref[3], params_ref[4], params_ref[5]
    deg = jnp.maximum(deg0_ref[...] + deg1_ref[...], 1.0)
    ld = jnp.log(deg)
    a1_ref[...] = s1 * jnp.exp(g1 * ld)
    b1_ref[...] = n1 * jnp.exp((g1 - 1.0) * ld)
    a2_ref[...] = s2 * jnp.exp(g2 * ld)
    b2_ref[...] = n2 * jnp.exp((g2 - 1.0) * ld)
    std_ref[...] = jax.nn.softplus(diag_ref[...])
    pdq_ref[...] = jax.nn.softplus(pdiag_ref[...])


def _std_body(diag_ref, std_ref):
    std_ref[...] = jax.nn.softplus(diag_ref[...])


def _tc_std(diag_pad, n_pad):
    m = n_pad // 128
    full = pl.BlockSpec((m, 128), lambda: (0, 0))
    out = pl.pallas_call(
        _std_body, in_specs=[full], out_specs=full,
        out_shape=jax.ShapeDtypeStruct((m, 128), jnp.float32),
    )(diag_pad.reshape(m, 128))
    return out.reshape(n_pad)


def _tc_factors(params, deg0, deg1, diag_pad, pdiag_pad, n_pad):
    m = n_pad // 128
    shp = jax.ShapeDtypeStruct((m, 128), jnp.float32)
    full = pl.BlockSpec((m, 128), lambda: (0, 0))
    outs = pl.pallas_call(
        _factor_body,
        in_specs=[pl.BlockSpec(memory_space=pltpu.SMEM)] + [full] * 4,
        out_specs=[full] * 6,
        out_shape=[shp] * 6,
    )(params, deg0.reshape(m, 128), deg1.reshape(m, 128),
      diag_pad.reshape(m, 128), pdiag_pad.reshape(m, 128))
    return [o.reshape(n_pad) for o in outs]


def kernel(standard_sample, mean_param, diag_param, post_diag_param,
           alpha1, alpha2, gamma_param, edge_index):
    S, N = standard_sample.shape
    E = edge_index.shape[1]
    bn = 2048
    n_pad = ((N + 1 + bn - 1) // bn) * bn
    e_align = _NW * _CHUNK * _WIN * 2
    e_pad = ((E + e_align - 1) // e_align) * e_align

    ss_t = jnp.pad(standard_sample.T, ((0, n_pad - N), (0, _LANES - S)))
    diag_pad = jnp.pad(diag_param, (0, n_pad - N))
    pdiag_pad = jnp.pad(post_diag_param, (0, n_pad - N))
    mean_pad = jnp.pad(mean_param, (0, n_pad - N))
    src2 = jnp.pad(edge_index[0], (0, e_pad - E),
                   constant_values=N).reshape(-1, _CHUNK)
    dst2 = jnp.pad(edge_index[1], (0, e_pad - E),
                   constant_values=N).reshape(-1, _CHUNK)
    sw = jnp.exp(alpha1)
    nw = sw * jnp.tanh(alpha2)
    g = jax.nn.sigmoid(gamma_param)
    params = jnp.stack([sw[0], nw[0], g[0], sw[1], nw[1], g[1]])

    stdf = _tc_std(diag_pad, n_pad)
    x0 = _sc_prep(ss_t, stdf, n_pad)
    raw1, deg0, deg1 = _sc_edge_pass(x0, src2, dst2, n_pad, e_pad,
                                     with_deg=True)
    _, a1, b1, a2, b2, pdq = _tc_factors(
        params, deg0, deg1, diag_pad, pdiag_pad, n_pad)
    x1 = _sc_combine(x0, raw1, a1, b1, n_pad)
    raw2 = _sc_edge_pass(x1, src2, dst2, n_pad, e_pad)
    out_t = _sc_final(x1, raw2, a2, b2, pdq, mean_pad, n_pad)
    return out_t[:N, :S].T


# speedup vs baseline: 97.9009x; 1.1106x over previous
import functools

import jax
import jax.numpy as jnp
from jax import lax
from jax.experimental import pallas as pl
from jax.experimental.pallas import tpu as pltpu
from jax.experimental.pallas import tpu_sc as plsc

_LANES = 16
_CHUNK = 256
_NC = 2
_NS = 16
_NW = _NC * _NS


_WIN = 4


def _sc_edge_pass_body(with_deg, n_pad, e_pad, *refs):
    if with_deg:
        (x_hbm, src_hbm, dst_hbm, zrows_hbm, zdeg_hbm,
         raw_hbm, deg0_hbm, deg1_hbm,
         acc_sh, deg_sh, idx_s, idx_d, rows_v, ones_v, *sems) = refs
    else:
        (x_hbm, src_hbm, dst_hbm, zrows_hbm,
         raw_hbm,
         acc_sh, idx_s, idx_d, rows_v, *sems) = refs
    si = sems[0:2]
    sg = sems[2:2 + _WIN]
    ss = sems[2 + _WIN:2 + 2 * _WIN]

    cid = lax.axis_index("c")
    sid = lax.axis_index("s")
    rpt = n_pad // _NS

    pltpu.sync_copy(zrows_hbm, acc_sh.at[pl.ds(sid * rpt, rpt)])
    if with_deg:
        pltpu.sync_copy(zdeg_hbm, deg_sh.at[pl.ds(sid * rpt, rpt)])
        for i in range(_CHUNK // 16):
            ones_v[pl.ds(i * 16, 16)] = jnp.ones((16,), jnp.float32)
    plsc.subcore_barrier()

    rows_per_sub = e_pad // (_NW * _CHUNK)
    nwin = rows_per_sub // _WIN
    base_row = (cid * _NS + sid) * rows_per_sub

    def fire_idx(h, w):
        r0 = base_row + w * _WIN
        pltpu.async_copy(src_hbm.at[pl.ds(r0, _WIN)], idx_s.at[h], si[h])
        pltpu.async_copy(dst_hbm.at[pl.ds(r0, _WIN)], idx_d.at[h], si[h])

    def wait_idx(h):
        r0 = base_row
        pltpu.make_async_copy(src_hbm.at[pl.ds(r0, _WIN)],
                              idx_s.at[h], si[h]).wait()
        pltpu.make_async_copy(dst_hbm.at[pl.ds(r0, _WIN)],
                              idx_d.at[h], si[h]).wait()

    def drain_bytes(j, sem):
        pltpu.make_async_copy(x_hbm.at[pl.ds(0, _CHUNK)],
                              rows_v.at[j], sem).wait()

    def drain_ones(j, sem):
        pltpu.make_async_copy(src_hbm.at[pl.ds(0, 1)],
                              idx_s.at[0, 0], sem).wait()

    def window(h, w):
        wait_idx(h)
        for j in range(_WIN):
            @pl.when(w >= 1)
            def _drain_ss():
                drain_bytes(j, ss[j])
                if with_deg:
                    drain_ones(j, ss[j])
            pltpu.async_copy(x_hbm.at[idx_s.at[h, j]], rows_v.at[j], sg[j])

        @pl.when(w + 1 < nwin)
        def _prefetch_idx():
            fire_idx(1 - h, w + 1)

        for j in range(_WIN):
            drain_bytes(j, sg[j])
            pltpu.async_copy(rows_v.at[j], acc_sh.at[idx_d.at[h, j]],
                             ss[j], add=True)
            if with_deg:
                pltpu.async_copy(ones_v, deg_sh.at[idx_s.at[h, j]],
                                 ss[j], add=True)

    fire_idx(0, 0)

    def body(g, carry):
        window(0, 2 * g)
        window(1, 2 * g + 1)
        return carry

    lax.fori_loop(0, nwin // 2, body, 0)
    for j in range(_WIN):
        drain_bytes(j, ss[j])
        if with_deg:
            drain_ones(j, ss[j])
    plsc.subcore_barrier()

    pltpu.sync_copy(acc_sh.at[pl.ds(sid * rpt, rpt)],
                    raw_hbm.at[cid, pl.ds(sid * rpt, rpt)])
    if with_deg:
        @pl.when(cid == 0)
        def _out0():
            pltpu.sync_copy(deg_sh.at[pl.ds(sid * rpt, rpt)],
                            deg0_hbm.at[pl.ds(sid * rpt, rpt)])

        @pl.when(cid == 1)
        def _out1():
            pltpu.sync_copy(deg_sh.at[pl.ds(sid * rpt, rpt)],
                            deg1_hbm.at[pl.ds(sid * rpt, rpt)])


def _sc_edge_pass(x_t, src2, dst2, n_pad, e_pad, with_deg=False):
    mesh = plsc.VectorSubcoreMesh(core_axis_name="c", subcore_axis_name="s")
    rpt = n_pad // _NS
    zrows = jnp.zeros((rpt, _LANES), jnp.float32)
    raw_t = jax.ShapeDtypeStruct((_NC, n_pad, _LANES), jnp.float32)
    idx_scr = [
        pltpu.VMEM((2, _WIN, _CHUNK), jnp.int32),
        pltpu.VMEM((2, _WIN, _CHUNK), jnp.int32),
        pltpu.VMEM((_WIN, _CHUNK, _LANES), jnp.float32),
    ]
    sems = [pltpu.SemaphoreType.DMA] * (2 + 2 * _WIN)
    if with_deg:
        out_type = (raw_t,
                    jax.ShapeDtypeStruct((n_pad,), jnp.float32),
                    jax.ShapeDtypeStruct((n_pad,), jnp.float32))
        scratch = ([pltpu.VMEM_SHARED((n_pad, _LANES), jnp.float32),
                    pltpu.VMEM_SHARED((n_pad,), jnp.float32)]
                   + idx_scr + [pltpu.VMEM((_CHUNK,), jnp.float32)] + sems)
        zdeg = jnp.zeros((rpt,), jnp.float32)
        args = (x_t, src2, dst2, zrows, zdeg)
    else:
        out_type = raw_t
        scratch = ([pltpu.VMEM_SHARED((n_pad, _LANES), jnp.float32)]
                   + idx_scr + sems)
        args = (x_t, src2, dst2, zrows)
    body = functools.partial(_sc_edge_pass_body, with_deg, n_pad, e_pad)
    return pl.kernel(
        body, out_type=out_type, mesh=mesh, scratch_types=scratch,
        compiler_params=pltpu.CompilerParams(use_tc_tiling_on_sc=False),
    )(*args)


_CR = 784
_IOTA = None


def _row_idx(r):
    return jnp.full((16,), r, jnp.int32), lax.iota(jnp.int32, 16)


def _sc_prep_body(n_pad, ss_hbm, std_hbm, x0_hbm, ssv, stdv, x0v):
    wid = lax.axis_index("c") * _NS + lax.axis_index("s")
    rows = n_pad // _NW
    base0 = wid * rows

    def chunk(ch, carry):
        base = base0 + ch * _CR
        pltpu.sync_copy(ss_hbm.at[pl.ds(base, _CR)], ssv)
        pltpu.sync_copy(std_hbm.at[pl.ds(base, _CR)], stdv)

        def row(r, c2):
            splat, io = _row_idx(r)
            f = plsc.load_gather(stdv, [splat])
            x = plsc.load_gather(ssv, [splat, io])
            plsc.store_scatter(x0v, [splat, io], f * x)
            return c2

        lax.fori_loop(0, _CR, row, 0, unroll=4)
        pltpu.sync_copy(x0v, x0_hbm.at[pl.ds(base, _CR)])
        return carry

    lax.fori_loop(0, rows // _CR, chunk, 0)


def _sc_prep(ss2, std, n_pad):
    mesh = plsc.VectorSubcoreMesh(core_axis_name="c", subcore_axis_name="s")
    out_type = jax.ShapeDtypeStruct((n_pad, _LANES), jnp.float32)
    scratch = [
        pltpu.VMEM((_CR, _LANES), jnp.float32),
        pltpu.VMEM((_CR,), jnp.float32),
        pltpu.VMEM((_CR, _LANES), jnp.float32),
    ]
    body = functools.partial(_sc_prep_body, n_pad)
    return pl.kernel(
        body, out_type=out_type, mesh=mesh, scratch_types=scratch,
        compiler_params=pltpu.CompilerParams(use_tc_tiling_on_sc=False,
                                            needs_layout_passes=False),
    )(ss2, std)


def _sc_combine_body(n_pad, x_hbm, raw_hbm, a_hbm, b_hbm, xn_hbm,
                     xv, r0v, r1v, av, bv, xnv):
    wid = lax.axis_index("c") * _NS + lax.axis_index("s")
    rows = n_pad // _NW
    base0 = wid * rows

    def chunk(ch, carry):
        base = base0 + ch * _CR
        pltpu.sync_copy(x_hbm.at[pl.ds(base, _CR)], xv)
        pltpu.sync_copy(raw_hbm.at[0, pl.ds(base, _CR)], r0v)
        pltpu.sync_copy(raw_hbm.at[1, pl.ds(base, _CR)], r1v)
        pltpu.sync_copy(a_hbm.at[pl.ds(base, _CR)], av)
        pltpu.sync_copy(b_hbm.at[pl.ds(base, _CR)], bv)

        def row(r, c2):
            splat, io = _row_idx(r)
            fa = plsc.load_gather(av, [splat])
            fb = plsc.load_gather(bv, [splat])
            x = plsc.load_gather(xv, [splat, io])
            r0 = plsc.load_gather(r0v, [splat, io])
            r1 = plsc.load_gather(r1v, [splat, io])
            plsc.store_scatter(xnv, [splat, io], fa * x + fb * (r0 + r1))
            return c2

        lax.fori_loop(0, _CR, row, 0, unroll=4)
        pltpu.sync_copy(xnv, xn_hbm.at[pl.ds(base, _CR)])
        return carry

    lax.fori_loop(0, rows // _CR, chunk, 0)


def _sc_combine(x, raw, a, b, n_pad):
    mesh = plsc.VectorSubcoreMesh(core_axis_name="c", subcore_axis_name="s")
    out_type = jax.ShapeDtypeStruct((n_pad, _LANES), jnp.float32)
    scratch = (
        [pltpu.VMEM((_CR, _LANES), jnp.float32)] * 3
        + [pltpu.VMEM((_CR,), jnp.float32)] * 2
        + [pltpu.VMEM((_CR, _LANES), jnp.float32)]
    )
    body = functools.partial(_sc_combine_body, n_pad)
    return pl.kernel(
        body, out_type=out_type, mesh=mesh, scratch_types=scratch,
        compiler_params=pltpu.CompilerParams(use_tc_tiling_on_sc=False,
                                            needs_layout_passes=False),
    )(x, raw, a, b)


def _sc_final_body(n_pad, x_hbm, raw_hbm, a_hbm, b_hbm, p_hbm, m_hbm, o_hbm,
                   xv, r0v, r1v, av, bv, pv, mv, ov):
    wid = lax.axis_index("c") * _NS + lax.axis_index("s")
    rows = n_pad // _NW
    base0 = wid * rows

    def chunk(ch, carry):
        base = base0 + ch * _CR
        pltpu.sync_copy(x_hbm.at[pl.ds(base, _CR)], xv)
        pltpu.sync_copy(raw_hbm.at[0, pl.ds(base, _CR)], r0v)
        pltpu.sync_copy(raw_hbm.at[1, pl.ds(base, _CR)], r1v)
        pltpu.sync_copy(a_hbm.at[pl.ds(base, _CR)], av)
        pltpu.sync_copy(b_hbm.at[pl.ds(base, _CR)], bv)
        pltpu.sync_copy(p_hbm.at[pl.ds(base, _CR)], pv)
        pltpu.sync_copy(m_hbm.at[pl.ds(base, _CR)], mv)

        def row(r, c2):
            splat, io = _row_idx(r)
            fa = plsc.load_gather(av, [splat])
            fb = plsc.load_gather(bv, [splat])
            fp = plsc.load_gather(pv, [splat])
            fm = plsc.load_gather(mv, [splat])
            x = plsc.load_gather(xv, [splat, io])
            r0 = plsc.load_gather(r0v, [splat, io])
            r1 = plsc.load_gather(r1v, [splat, io])
            x2 = fa * x + fb * (r0 + r1)
            plsc.store_scatter(ov, [io, splat], fp * x2 + fm)
            return c2

        lax.fori_loop(0, _CR, row, 0, unroll=4)
        pltpu.sync_copy(ov, o_hbm.at[:, pl.ds(base, _CR)])
        return carry

    lax.fori_loop(0, rows // _CR, chunk, 0)


def _sc_final(x, raw, a, b, p, m, n_pad):
    mesh = plsc.VectorSubcoreMesh(core_axis_name="c", subcore_axis_name="s")
    out_type = jax.ShapeDtypeStruct((_LANES, n_pad), jnp.float32)
    scratch = (
        [pltpu.VMEM((_CR, _LANES), jnp.float32)] * 3
        + [pltpu.VMEM((_CR,), jnp.float32)] * 4
        + [pltpu.VMEM((_LANES, _CR), jnp.float32)]
    )
    body = functools.partial(_sc_final_body, n_pad)
    return pl.kernel(
        body, out_type=out_type, mesh=mesh, scratch_types=scratch,
        compiler_params=pltpu.CompilerParams(use_tc_tiling_on_sc=False,
                                            needs_layout_passes=False),
    )(x, raw, a, b, p, m)


def _factor_body(params_ref, deg0_ref, deg1_ref, diag_ref, pdiag_ref,
                 std_ref, a1_ref, b1_ref, a2_ref, b2_ref, pdq_ref):
    s1, n1, g1 = params_ref[0], params_ref[1], params_ref[2]
    s2, n2, g2 = params_ref[3], params_ref[4], params_ref[5]
    deg = jnp.maximum(deg0_ref[...] + deg1_ref[...], 1.0)
    ld = jnp.log(deg)
    a1_ref[...] = s1 * jnp.exp(g1 * ld)
    b1_ref[...] = n1 * jnp.exp((g1 - 1.0) * ld)
    a2_ref[...] = s2 * jnp.exp(g2 * ld)
    b2_ref[...] = n2 * jnp.exp((g2 - 1.0) * ld)
    std_ref[...] = jax.nn.softplus(diag_ref[...])
    pdq_ref[...] = jax.nn.softplus(pdiag_ref[...])


def _std_body(diag_ref, std_ref):
    std_ref[...] = jax.nn.softplus(diag_ref[...])


def _tc_std(diag_pad, n_pad):
    m = n_pad // 128
    full = pl.BlockSpec((m, 128), lambda: (0, 0))
    out = pl.pallas_call(
        _std_body, in_specs=[full], out_specs=full,
        out_shape=jax.ShapeDtypeStruct((m, 128), jnp.float32),
    )(diag_pad.reshape(m, 128))
    return out.reshape(n_pad)


def _tc_factors(params, deg0, deg1, diag_pad, pdiag_pad, n_pad):
    m = n_pad // 128
    shp = jax.ShapeDtypeStruct((m, 128), jnp.float32)
    full = pl.BlockSpec((m, 128), lambda: (0, 0))
    outs = pl.pallas_call(
        _factor_body,
        in_specs=[pl.BlockSpec(memory_space=pltpu.SMEM)] + [full] * 4,
        out_specs=[full] * 6,
        out_shape=[shp] * 6,
    )(params, deg0.reshape(m, 128), deg1.reshape(m, 128),
      diag_pad.reshape(m, 128), pdiag_pad.reshape(m, 128))
    return [o.reshape(n_pad) for o in outs]


def kernel(standard_sample, mean_param, diag_param, post_diag_param,
           alpha1, alpha2, gamma_param, edge_index):
    S, N = standard_sample.shape
    E = edge_index.shape[1]
    bn = 2048
    n_pad = ((N + 1 + bn - 1) // bn) * bn
    e_align = _NW * _CHUNK * _WIN * 2
    e_pad = ((E + e_align - 1) // e_align) * e_align

    ss_t = jnp.pad(standard_sample.T, ((0, n_pad - N), (0, _LANES - S)))
    diag_pad = jnp.pad(diag_param, (0, n_pad - N))
    pdiag_pad = jnp.pad(post_diag_param, (0, n_pad - N))
    mean_pad = jnp.pad(mean_param, (0, n_pad - N))
    src2 = jnp.pad(edge_index[0], (0, e_pad - E),
                   constant_values=N).reshape(-1, _CHUNK)
    dst2 = jnp.pad(edge_index[1], (0, e_pad - E),
                   constant_values=N).reshape(-1, _CHUNK)
    sw = jnp.exp(alpha1)
    nw = sw * jnp.tanh(alpha2)
    g = jax.nn.sigmoid(gamma_param)
    params = jnp.stack([sw[0], nw[0], g[0], sw[1], nw[1], g[1]])

    stdf = _tc_std(diag_pad, n_pad)
    x0 = _sc_prep(ss_t, stdf, n_pad)
    raw1, deg0, deg1 = _sc_edge_pass(x0, src2, dst2, n_pad, e_pad,
                                     with_deg=True)
    _, a1, b1, a2, b2, pdq = _tc_factors(
        params, deg0, deg1, diag_pad, pdiag_pad, n_pad)
    x1 = _sc_combine(x0, raw1, a1, b1, n_pad)
    raw2 = _sc_edge_pass(x1, src2, dst2, n_pad, e_pad)
    out_t = _sc_final(x1, raw2, a2, b2, pdq, mean_pad, n_pad)
    return out_t[:S, :N]
